# baseline (speedup 1.0000x reference)
import functools

import jax
import jax.numpy as jnp
from jax import lax
from jax.experimental import pallas as pl
from jax.experimental.pallas import tpu as pltpu
from jax.experimental.pallas import tpu_sc as plsc

NC = 2
NS = 16
LN = 16
TILES = NC * NS

F = 64
NPT = 320
DUMROW = NPT
NEG = float("-inf")

_MESH = plsc.VectorSubcoreMesh(
    core_axis_name="c", subcore_axis_name="s", num_cores=NC, num_subcores=NS)
_CP = pltpu.CompilerParams(needs_layout_passes=False)

CH = 2560


def _wid():
    return lax.axis_index("s") * NC + lax.axis_index("c")


def _iota():
    return lax.iota(jnp.int32, LN)


def _kern(out_type, scratch):
    return functools.partial(pl.kernel, out_type=out_type, mesh=_MESH,
                             scratch_types=scratch, compiler_params=_CP)


def _my_start(counts_vm, w):
    c0 = plsc.load_gather(counts_vm, [_iota() * LN])
    c1 = plsc.load_gather(counts_vm, [(LN + _iota()) * LN])
    i = _iota()
    s = (jnp.sum(jnp.where(i < w, c0, 0))
         + jnp.sum(jnp.where(i + LN < w, c1, 0)))
    mynb = jnp.max(plsc.load_gather(counts_vm, [jnp.broadcast_to(w * LN, (LN,))]))
    return s, mynb


def _count(dst, E):
    @_kern(jax.ShapeDtypeStruct((TILES * LN,), jnp.int32),
           [pltpu.VMEM((CH,), jnp.int32), pltpu.VMEM((LN,), jnp.int32)])
    def k(dst_hbm, counts_hbm, dvm, cvb):
        w = _wid()
        lo = w * NPT
        hi = lo + NPT

        def chunk(ch, cv):
            pltpu.sync_copy(dst_hbm.at[pl.ds(ch * CH, CH)], dvm)
            for v in range(CH // LN):
                d = dvm[pl.ds(v * LN, LN)]
                cv = cv + ((d >= lo) & (d < hi)).astype(jnp.int32)
            return cv

        cv = pl.loop(0, E // CH, init_carry=jnp.zeros((LN,), jnp.int32))(chunk)
        nb = (jnp.sum(cv) + 127) // 128
        cvb[...] = jnp.broadcast_to(nb, (LN,))
        pltpu.sync_copy(cvb, counts_hbm.at[pl.ds(w * LN, LN)])

    return k(dst)


def _compact(dst, src, counts, E, PADE):
    STG = 4096

    @_kern([jax.ShapeDtypeStruct((PADE,), jnp.int32),
            jax.ShapeDtypeStruct((PADE,), jnp.int32)],
           [pltpu.VMEM((CH,), jnp.int32), pltpu.VMEM((CH,), jnp.int32),
            pltpu.VMEM((STG,), jnp.int32), pltpu.VMEM((STG,), jnp.int32),
            pltpu.VMEM((TILES * LN,), jnp.int32)])
    def k(dst_hbm, src_hbm, counts_hbm, dl_hbm, sl_hbm, dvm, svm, dstg, sstg,
          cvm):
        w = _wid()
        lo = w * NPT
        hi = lo + NPT
        iota = _iota()
        pltpu.sync_copy(counts_hbm, cvm)
        sb, _ = _my_start(cvm, w)
        base = pl.multiple_of(sb * 128, 128)

        def chunk(ch, carry):
            off, nfl = carry
            pltpu.sync_copy(dst_hbm.at[pl.ds(ch * CH, CH)], dvm)
            pltpu.sync_copy(src_hbm.at[pl.ds(ch * CH, CH)], svm)
            for v in range(CH // LN):
                d = dvm[pl.ds(v * LN, LN)]
                s = svm[pl.ds(v * LN, LN)]
                msk = (d >= lo) & (d < hi)
                cum = plsc.cumsum(msk.astype(jnp.int32))
                pos = (off + cum - 1) & (STG - 1)
                plsc.store_scatter(dstg, [pos], d - lo, mask=msk)
                plsc.store_scatter(sstg, [pos], s, mask=msk)
                off = off + jnp.sum(msk.astype(jnp.int32))

            def fl1024(kf, _):
                so = pl.multiple_of((kf * 1024) & (STG - 1), 1024)
                o = pl.multiple_of(base + kf * 1024, 128)
                pltpu.sync_copy(dstg.at[pl.ds(so, 1024)],
                                dl_hbm.at[pl.ds(o, 1024)])
                pltpu.sync_copy(sstg.at[pl.ds(so, 1024)],
                                sl_hbm.at[pl.ds(o, 1024)])
                return 0

            nfl_new = off // 1024
            pl.loop(nfl, nfl_new, init_carry=0)(fl1024)
            return off, nfl_new

        off, nfl = pl.loop(0, E // CH,
                           init_carry=(jnp.int32(0), jnp.int32(0)))(chunk)

        tgt = (off + 127) & (-128)
        dumd = jnp.full((LN,), DUMROW, jnp.int32)
        zero = jnp.zeros((LN,), jnp.int32)
        for kp in range(8):
            adr = off + kp * LN + iota
            m = adr < tgt
            pos = adr & (STG - 1)
            plsc.store_scatter(dstg, [pos], dumd, mask=m)
            plsc.store_scatter(sstg, [pos], zero, mask=m)

        def fl(q, _):
            eo = nfl * 1024 + q * 128
            so = pl.multiple_of(eo & (STG - 1), 128)
            o = pl.multiple_of(base + eo, 128)
            pltpu.sync_copy(dstg.at[pl.ds(so, 128)], dl_hbm.at[pl.ds(o, 128)])
            pltpu.sync_copy(sstg.at[pl.ds(so, 128)], sl_hbm.at[pl.ds(o, 128)])
            return 0

        pl.loop(0, (tgt - nfl * 1024) // 128, init_carry=0)(fl)

    return k(dst, src, counts)


def _edge_pre(A, Bpad, dlocl, srcl, counts, PADN, PADE):
    BT = 128

    @_kern(jax.ShapeDtypeStruct((PADE, F), jnp.float32),
           [pltpu.VMEM((NPT, F), jnp.float32),
            pltpu.VMEM((BT,), jnp.int32),
            pltpu.VMEM((BT,), jnp.int32),
            pltpu.VMEM((BT, 2 * F), jnp.float32),
            pltpu.VMEM((BT, F), jnp.float32),
            pltpu.VMEM((TILES * LN,), jnp.int32),
            pltpu.SemaphoreType.DMA])
    def k(a_hbm, b_hbm, dl_hbm, sl_hbm, counts_hbm, pre_hbm, al, dlv, siv,
          brows, ost, cvm, sem):
        w = _wid()
        iota = _iota()
        pltpu.sync_copy(counts_hbm, cvm)
        sb, mynb = _my_start(cvm, w)
        base = sb * 128
        pltpu.sync_copy(a_hbm.at[pl.ds(w * NPT, NPT), :], al)

        def body(b, _):
            e0 = base + b * BT
            pltpu.sync_copy(dl_hbm.at[pl.ds(e0, BT)], dlv)
            pltpu.sync_copy(sl_hbm.at[pl.ds(e0, BT)], siv)
            pltpu.async_copy(b_hbm.at[siv], brows, sem).wait()
            for g in range(BT // LN):
                jv = g * LN + iota
                dl = dlv[pl.ds(g * LN, LN)]
                for f in range(F):
                    fv = jnp.full((LN,), f, jnp.int32)
                    av = plsc.load_gather(al, [dl, fv])
                    bv = plsc.load_gather(brows, [jv, fv])
                    plsc.store_scatter(ost, [jv, fv], av + bv)
            pltpu.sync_copy(ost, pre_hbm.at[pl.ds(e0, BT), :])
            return 0

        pl.loop(0, mynb, init_carry=0)(body)

    return k(A, Bpad, dlocl, srcl, counts)


def _scatter_max(m, dlocl, counts, PADN, PADE):
    BT = 128
    AF = (NPT + 1) * F

    @_kern(jax.ShapeDtypeStruct((PADN * F,), jnp.float32),
           [pltpu.VMEM((BT,), jnp.int32),
            pltpu.VMEM((BT, F), jnp.float32),
            pltpu.VMEM((AF,), jnp.float32),
            pltpu.VMEM((TILES * LN,), jnp.int32)])
    def k(m_hbm, dl_hbm, counts_hbm, h_hbm, dlv, rows, acc, cvm):
        w = _wid()
        iota = _iota()
        pltpu.sync_copy(counts_hbm, cvm)
        sb, mynb = _my_start(cvm, w)
        base = sb * 128
        negv = jnp.full((LN,), NEG, jnp.float32)

        def init(i, _):
            plsc.store_scatter(acc, [i * LN + iota], negv)
            return 0

        pl.loop(0, AF // LN, init_carry=0)(init)

        def body(b, _):
            e0 = base + b * BT
            pltpu.sync_copy(dl_hbm.at[pl.ds(e0, BT)], dlv)
            pltpu.sync_copy(m_hbm.at[pl.ds(e0, BT), :], rows)

            def edge(j, _c):
                for u in range(4):
                    jv = jnp.broadcast_to(j + u, (LN,))
                    dl = plsc.load_gather(dlv, [jv])
                    for q in range(F // LN):
                        ci = q * LN + iota
                        rv = plsc.load_gather(rows, [jv, ci])
                        adr = dl * F + ci
                        av = plsc.load_gather(acc, [adr])
                        plsc.store_scatter(acc, [adr], jnp.maximum(av, rv))
                return 0

            pl.loop(0, BT, init_carry=0, step=4)(edge)
            return 0

        pl.loop(0, mynb, init_carry=0)(body)

        def fix(i, _):
            adr = i * LN + iota
            v = plsc.load_gather(acc, [adr])
            plsc.store_scatter(acc, [adr], jnp.where(v == NEG, 0.0, v))
            return 0

        pl.loop(0, NPT * F // LN, init_carry=0)(fix)
        pltpu.sync_copy(acc.at[pl.ds(0, NPT * F)],
                        h_hbm.at[pl.ds(w * NPT * F, NPT * F)])

    return k(m, dlocl, counts)


def _pool_sc(hflat, batch_pad, G, PADN):
    NG = G + 1
    PF = NG * F

    @_kern([jax.ShapeDtypeStruct((TILES * PF,), jnp.float32),
            jax.ShapeDtypeStruct((TILES * PF,), jnp.float32)],
           [pltpu.VMEM((NPT * F,), jnp.float32),
            pltpu.VMEM((NPT,), jnp.int32),
            pltpu.VMEM((PF,), jnp.float32),
            pltpu.VMEM((PF,), jnp.float32)])
    def k(h_hbm, b_hbm, psum_hbm, pmax_hbm, hv, bv, ps, pm):
        w = _wid()
        iota = _iota()
        pltpu.sync_copy(h_hbm.at[pl.ds(w * NPT * F, NPT * F)], hv)
        pltpu.sync_copy(b_hbm.at[pl.ds(w * NPT, NPT)], bv)
        negv = jnp.full((LN,), NEG, jnp.float32)
        zv = jnp.zeros((LN,), jnp.float32)

        def init(i, _):
            plsc.store_scatter(ps, [i * LN + iota], zv)
            plsc.store_scatter(pm, [i * LN + iota], negv)
            return 0

        pl.loop(0, PF // LN, init_carry=0)(init)

        def node(j, _c):
            jv = jnp.broadcast_to(j, (LN,))
            g = plsc.load_gather(bv, [jv])
            for q in range(F // LN):
                ci = q * LN + iota
                val = plsc.load_gather(hv, [jv * F + ci])
                adr = g * F + ci
                cs = plsc.load_gather(ps, [adr])
                plsc.store_scatter(ps, [adr], cs + val)
                cm = plsc.load_gather(pm, [adr])
                plsc.store_scatter(pm, [adr], jnp.maximum(cm, val))
            return 0

        pl.loop(0, NPT, init_carry=0)(node)
        pltpu.sync_copy(ps, psum_hbm.at[pl.ds(w * PF, PF)])
        pltpu.sync_copy(pm, pmax_hbm.at[pl.ds(w * PF, PF)])

    return k(hflat, batch_pad)


def _node_ab(h, W1, b1, Din, PADN):
    BR = PADN // 8

    def body(h_ref, w_ref, b_ref, a_ref, bo_ref):
        wa = w_ref[0:Din, :]
        wb = w_ref[Din:2 * Din, :]
        hv = h_ref[...]
        a_ref[...] = jnp.dot(hv, wa - wb,
                             preferred_element_type=jnp.float32,
                             precision=lax.Precision.HIGHEST) + b_ref[...]
        hb = jnp.dot(hv, wb, preferred_element_type=jnp.float32,
                     precision=lax.Precision.HIGHEST)
        bo_ref[...] = jnp.concatenate(
            [hb, jnp.zeros((BR, F), jnp.float32)], axis=1)

    return pl.pallas_call(
        body,
        grid=(8,),
        in_specs=[
            pl.BlockSpec((BR, Din), lambda i: (i, 0)),
            pl.BlockSpec((2 * Din, F), lambda i: (0, 0)),
            pl.BlockSpec((1, F), lambda i: (0, 0)),
        ],
        out_specs=[
            pl.BlockSpec((BR, F), lambda i: (i, 0)),
            pl.BlockSpec((BR, 2 * F), lambda i: (i, 0)),
        ],
        out_shape=[
            jax.ShapeDtypeStruct((PADN, F), jnp.float32),
            jax.ShapeDtypeStruct((PADN, 2 * F), jnp.float32),
        ],
    )(h, W1, b1)


def _edge_mlp(pre, W2, b2, PADE):
    BR = 1280

    def body(p_ref, w_ref, b_ref, o_ref):
        mm = jnp.maximum(p_ref[...], 0.0)
        o_ref[...] = jnp.dot(mm, w_ref[...],
                             preferred_element_type=jnp.float32,
                             precision=lax.Precision.HIGHEST) + b_ref[...]

    return pl.pallas_call(
        body,
        grid=(PADE // BR,),
        in_specs=[
            pl.BlockSpec((BR, F), lambda i: (i, 0)),
            pl.BlockSpec((F, F), lambda i: (0, 0)),
            pl.BlockSpec((1, F), lambda i: (0, 0)),
        ],
        out_specs=pl.BlockSpec((BR, F), lambda i: (i, 0)),
        out_shape=jax.ShapeDtypeStruct((PADE, F), jnp.float32),
    )(pre, W2, b2)


def _classifier(psum, pmax, batch2d, cw1, cb1, cw2, cb2, cw3, cb3, G):
    NG = G + 1
    NBR, NBC = batch2d.shape

    def body(ps_ref, pm_ref, bt_ref, w1_ref, b1_ref, w2_ref, b2_ref, w3_ref,
             b3_ref, o_ref):
        s = jnp.sum(ps_ref[...], axis=0)[0:G, :]
        mx = jnp.max(pm_ref[...], axis=0)[0:G, :]
        mx = jnp.where(mx == NEG, 0.0, mx)
        counts = jnp.zeros((G, 1), jnp.float32)
        giota = lax.broadcasted_iota(jnp.int32, (G, NBC), 0)
        for i in range(NBR):
            bi = bt_ref[i:i + 1, :]
            oh = (jnp.broadcast_to(bi, (G, NBC)) == giota)
            counts = counts + jnp.sum(oh.astype(jnp.float32), axis=1,
                                      keepdims=True)
        x_mean = s / jnp.maximum(counts, 1.0)
        z = jnp.concatenate([x_mean, mx], axis=1)
        z = jnp.maximum(jnp.dot(z, w1_ref[...],
                                preferred_element_type=jnp.float32,
                                precision=lax.Precision.HIGHEST)
                        + b1_ref[...], 0.0)
        z = jnp.maximum(jnp.dot(z, w2_ref[...],
                                preferred_element_type=jnp.float32,
                                precision=lax.Precision.HIGHEST)
                        + b2_ref[...], 0.0)
        o_ref[...] = jnp.dot(z, w3_ref[...],
                             preferred_element_type=jnp.float32,
                             precision=lax.Precision.HIGHEST) + b3_ref[...]

    def full(shape):
        return pl.BlockSpec(shape, lambda: tuple(0 for _ in shape))

    return pl.pallas_call(
        body,
        in_specs=[
            full((TILES, NG, F)),
            full((TILES, NG, F)),
            full((NBR, NBC)),
            full((2 * F, F)),
            full((1, F)),
            full((F, F)),
            full((1, F)),
            full((F, 128)),
            full((1, 128)),
        ],
        out_specs=full((G, 128)),
        out_shape=jax.ShapeDtypeStruct((G, 128), jnp.float32),
    )(psum, pmax, batch2d, cw1, cb1, cw2, cb2, cw3, cb3)


def kernel(x, edge_index, batch, params):
    N, D = x.shape
    E = edge_index.shape[1]
    G = 64
    PADN = TILES * NPT
    PADE = E + 5120

    src = edge_index[0]
    dst = edge_index[1]
    xp = jnp.pad(x, ((0, PADN - N), (0, 0)))
    batch_pad = jnp.pad(batch, (0, PADN - N), constant_values=G)

    counts = _count(dst, E)
    dlocl, srcl = _compact(dst, src, counts, E, PADE)

    h = xp
    din = D
    for i in range(3):
        W1 = params[f"conv{i}_W1"]
        b1 = params[f"conv{i}_b1"].reshape(1, F)
        W2 = params[f"conv{i}_W2"]
        b2 = params[f"conv{i}_b2"].reshape(1, F)
        A, Bpad = _node_ab(h, W1, b1, din, PADN)
        pre = _edge_pre(A, Bpad, dlocl, srcl, counts, PADN, PADE)
        m = _edge_mlp(pre, W2, b2, PADE)
        hflat = _scatter_max(m, dlocl, counts, PADN, PADE)
        h = hflat.reshape(PADN, F)
        din = F

    psum, pmax = _pool_sc(hflat, batch_pad, G, PADN)
    NG = G + 1
    psum3 = psum.reshape(TILES, NG, F)
    pmax3 = pmax.reshape(TILES, NG, F)

    batch2d = batch_pad.reshape(16, PADN // 16)
    cw1 = params["cls_W1"]
    cb1 = params["cls_b1"].reshape(1, F)
    cw2 = params["cls_W2"]
    cb2 = params["cls_b2"].reshape(1, F)
    cw3 = jnp.pad(params["cls_W3"], ((0, 0), (0, 127)))
    cb3 = jnp.pad(params["cls_b3"].reshape(1, 1), ((0, 0), (0, 127)))
    out = _classifier(psum3, pmax3, batch2d, cw1, cb1, cw2, cb2, cw3, cb3, G)
    return out[:, 0]


# speedup vs baseline: 1.5592x; 1.5592x over previous
import functools

import jax
import jax.numpy as jnp
from jax import lax
from jax.experimental import pallas as pl
from jax.experimental.pallas import tpu as pltpu
from jax.experimental.pallas import tpu_sc as plsc

NC = 2
NS = 16
LN = 16
TILES = NC * NS

F = 64
NPT = 320
DUMROW = NPT
NEG = float("-inf")

_MESH = plsc.VectorSubcoreMesh(
    core_axis_name="c", subcore_axis_name="s", num_cores=NC, num_subcores=NS)
_CP = pltpu.CompilerParams(needs_layout_passes=False)

CH = 2560


def _wid():
    return lax.axis_index("s") * NC + lax.axis_index("c")


def _iota():
    return lax.iota(jnp.int32, LN)


def _kern(out_type, scratch):
    return functools.partial(pl.kernel, out_type=out_type, mesh=_MESH,
                             scratch_types=scratch, compiler_params=_CP)


def _my_start(counts_vm, w):
    c0 = plsc.load_gather(counts_vm, [_iota() * LN])
    c1 = plsc.load_gather(counts_vm, [(LN + _iota()) * LN])
    i = _iota()
    s = (jnp.sum(jnp.where(i < w, c0, 0))
         + jnp.sum(jnp.where(i + LN < w, c1, 0)))
    mynb = jnp.max(plsc.load_gather(counts_vm, [jnp.broadcast_to(w * LN, (LN,))]))
    return s, mynb


def _count(dst, E):
    @_kern(jax.ShapeDtypeStruct((TILES * LN,), jnp.int32),
           [pltpu.VMEM((CH,), jnp.int32), pltpu.VMEM((LN,), jnp.int32)])
    def k(dst_hbm, counts_hbm, dvm, cvb):
        w = _wid()
        lo = w * NPT
        hi = lo + NPT

        def chunk(ch, cv):
            pltpu.sync_copy(dst_hbm.at[pl.ds(ch * CH, CH)], dvm)
            for v in range(CH // LN):
                d = dvm[pl.ds(v * LN, LN)]
                cv = cv + ((d >= lo) & (d < hi)).astype(jnp.int32)
            return cv

        cv = pl.loop(0, E // CH, init_carry=jnp.zeros((LN,), jnp.int32))(chunk)
        nb = (jnp.sum(cv) + 127) // 128
        cvb[...] = jnp.broadcast_to(nb, (LN,))
        pltpu.sync_copy(cvb, counts_hbm.at[pl.ds(w * LN, LN)])

    return k(dst)


def _compact(dst, src, counts, E, PADE):
    STG = 4096

    @_kern([jax.ShapeDtypeStruct((PADE,), jnp.int32),
            jax.ShapeDtypeStruct((PADE,), jnp.int32)],
           [pltpu.VMEM((CH,), jnp.int32), pltpu.VMEM((CH,), jnp.int32),
            pltpu.VMEM((STG,), jnp.int32), pltpu.VMEM((STG,), jnp.int32),
            pltpu.VMEM((TILES * LN,), jnp.int32)])
    def k(dst_hbm, src_hbm, counts_hbm, dl_hbm, sl_hbm, dvm, svm, dstg, sstg,
          cvm):
        w = _wid()
        lo = w * NPT
        hi = lo + NPT
        iota = _iota()
        pltpu.sync_copy(counts_hbm, cvm)
        sb, _ = _my_start(cvm, w)
        base = pl.multiple_of(sb * 128, 128)

        def chunk(ch, carry):
            off, nfl = carry
            pltpu.sync_copy(dst_hbm.at[pl.ds(ch * CH, CH)], dvm)
            pltpu.sync_copy(src_hbm.at[pl.ds(ch * CH, CH)], svm)
            for v in range(CH // LN):
                d = dvm[pl.ds(v * LN, LN)]
                s = svm[pl.ds(v * LN, LN)]
                msk = (d >= lo) & (d < hi)
                cum = plsc.cumsum(msk.astype(jnp.int32))
                pos = (off + cum - 1) & (STG - 1)
                plsc.store_scatter(dstg, [pos], d - lo, mask=msk)
                plsc.store_scatter(sstg, [pos], s, mask=msk)
                off = off + jnp.sum(msk.astype(jnp.int32))

            def fl1024(kf, _):
                so = pl.multiple_of((kf * 1024) & (STG - 1), 1024)
                o = pl.multiple_of(base + kf * 1024, 128)
                pltpu.sync_copy(dstg.at[pl.ds(so, 1024)],
                                dl_hbm.at[pl.ds(o, 1024)])
                pltpu.sync_copy(sstg.at[pl.ds(so, 1024)],
                                sl_hbm.at[pl.ds(o, 1024)])
                return 0

            nfl_new = off // 1024
            pl.loop(nfl, nfl_new, init_carry=0)(fl1024)
            return off, nfl_new

        off, nfl = pl.loop(0, E // CH,
                           init_carry=(jnp.int32(0), jnp.int32(0)))(chunk)

        tgt = (off + 127) & (-128)
        dumd = jnp.full((LN,), DUMROW, jnp.int32)
        zero = jnp.zeros((LN,), jnp.int32)
        for kp in range(8):
            adr = off + kp * LN + iota
            m = adr < tgt
            pos = adr & (STG - 1)
            plsc.store_scatter(dstg, [pos], dumd, mask=m)
            plsc.store_scatter(sstg, [pos], zero, mask=m)

        def fl(q, _):
            eo = nfl * 1024 + q * 128
            so = pl.multiple_of(eo & (STG - 1), 128)
            o = pl.multiple_of(base + eo, 128)
            pltpu.sync_copy(dstg.at[pl.ds(so, 128)], dl_hbm.at[pl.ds(o, 128)])
            pltpu.sync_copy(sstg.at[pl.ds(so, 128)], sl_hbm.at[pl.ds(o, 128)])
            return 0

        pl.loop(0, (tgt - nfl * 1024) // 128, init_carry=0)(fl)

    return k(dst, src, counts)


def _edge_pre(A, Bpad, dlocl, srcl, counts, PADN, PADE):
    BT = 128

    @_kern(jax.ShapeDtypeStruct((PADE, F), jnp.float32),
           [pltpu.VMEM((NPT, F), jnp.float32),
            pltpu.VMEM((BT,), jnp.int32),
            pltpu.VMEM((BT,), jnp.int32),
            pltpu.VMEM((BT, 2 * F), jnp.float32),
            pltpu.VMEM((BT, F), jnp.float32),
            pltpu.VMEM((TILES * LN,), jnp.int32),
            pltpu.SemaphoreType.DMA])
    def k(a_hbm, b_hbm, dl_hbm, sl_hbm, counts_hbm, pre_hbm, al, dlv, siv,
          brows, ost, cvm, sem):
        w = _wid()
        iota = _iota()
        pltpu.sync_copy(counts_hbm, cvm)
        sb, mynb = _my_start(cvm, w)
        base = sb * 128
        pltpu.sync_copy(a_hbm.at[pl.ds(w * NPT, NPT), :], al)

        def body(b, _):
            e0 = base + b * BT
            pltpu.sync_copy(dl_hbm.at[pl.ds(e0, BT)], dlv)
            pltpu.sync_copy(sl_hbm.at[pl.ds(e0, BT)], siv)
            pltpu.async_copy(b_hbm.at[siv], brows, sem).wait()

            def edge(j, _c):
                for u in range(4):
                    jv = jnp.broadcast_to(j + u, (LN,))
                    dl = plsc.load_gather(dlv, [jv])
                    for q in range(F // LN):
                        ci = q * LN + iota
                        av = plsc.load_gather(al, [dl, ci])
                        bv = plsc.load_gather(brows, [jv, ci])
                        plsc.store_scatter(ost, [jv, ci], av + bv)
                return 0

            pl.loop(0, BT, init_carry=0, step=4)(edge)
            pltpu.sync_copy(ost, pre_hbm.at[pl.ds(e0, BT), :])
            return 0

        pl.loop(0, mynb, init_carry=0)(body)

    return k(A, Bpad, dlocl, srcl, counts)


def _scatter_max(m, dlocl, counts, PADN, PADE):
    BT = 128
    AF = (NPT + 1) * F

    @_kern(jax.ShapeDtypeStruct((PADN * F,), jnp.float32),
           [pltpu.VMEM((BT,), jnp.int32),
            pltpu.VMEM((BT, F), jnp.float32),
            pltpu.VMEM((AF,), jnp.float32),
            pltpu.VMEM((TILES * LN,), jnp.int32)])
    def k(m_hbm, dl_hbm, counts_hbm, h_hbm, dlv, rows, acc, cvm):
        w = _wid()
        iota = _iota()
        pltpu.sync_copy(counts_hbm, cvm)
        sb, mynb = _my_start(cvm, w)
        base = sb * 128
        negv = jnp.full((LN,), NEG, jnp.float32)

        def init(i, _):
            plsc.store_scatter(acc, [i * LN + iota], negv)
            return 0

        pl.loop(0, AF // LN, init_carry=0)(init)

        def body(b, _):
            e0 = base + b * BT
            pltpu.sync_copy(dl_hbm.at[pl.ds(e0, BT)], dlv)
            pltpu.sync_copy(m_hbm.at[pl.ds(e0, BT), :], rows)

            def edge(j, _c):
                for u in range(4):
                    jv = jnp.broadcast_to(j + u, (LN,))
                    dl = plsc.load_gather(dlv, [jv])
                    for q in range(F // LN):
                        ci = q * LN + iota
                        rv = plsc.load_gather(rows, [jv, ci])
                        adr = dl * F + ci
                        av = plsc.load_gather(acc, [adr])
                        plsc.store_scatter(acc, [adr], jnp.maximum(av, rv))
                return 0

            pl.loop(0, BT, init_carry=0, step=4)(edge)
            return 0

        pl.loop(0, mynb, init_carry=0)(body)

        def fix(i, _):
            adr = i * LN + iota
            v = plsc.load_gather(acc, [adr])
            plsc.store_scatter(acc, [adr], jnp.where(v == NEG, 0.0, v))
            return 0

        pl.loop(0, NPT * F // LN, init_carry=0)(fix)
        pltpu.sync_copy(acc.at[pl.ds(0, NPT * F)],
                        h_hbm.at[pl.ds(w * NPT * F, NPT * F)])

    return k(m, dlocl, counts)


def _pool_sc(hflat, batch_pad, G, PADN):
    NG = G + 1
    PF = NG * F

    @_kern([jax.ShapeDtypeStruct((TILES * PF,), jnp.float32),
            jax.ShapeDtypeStruct((TILES * PF,), jnp.float32)],
           [pltpu.VMEM((NPT * F,), jnp.float32),
            pltpu.VMEM((NPT,), jnp.int32),
            pltpu.VMEM((PF,), jnp.float32),
            pltpu.VMEM((PF,), jnp.float32)])
    def k(h_hbm, b_hbm, psum_hbm, pmax_hbm, hv, bv, ps, pm):
        w = _wid()
        iota = _iota()
        pltpu.sync_copy(h_hbm.at[pl.ds(w * NPT * F, NPT * F)], hv)
        pltpu.sync_copy(b_hbm.at[pl.ds(w * NPT, NPT)], bv)
        negv = jnp.full((LN,), NEG, jnp.float32)
        zv = jnp.zeros((LN,), jnp.float32)

        def init(i, _):
            plsc.store_scatter(ps, [i * LN + iota], zv)
            plsc.store_scatter(pm, [i * LN + iota], negv)
            return 0

        pl.loop(0, PF // LN, init_carry=0)(init)

        def node(j, _c):
            jv = jnp.broadcast_to(j, (LN,))
            g = plsc.load_gather(bv, [jv])
            for q in range(F // LN):
                ci = q * LN + iota
                val = plsc.load_gather(hv, [jv * F + ci])
                adr = g * F + ci
                cs = plsc.load_gather(ps, [adr])
                plsc.store_scatter(ps, [adr], cs + val)
                cm = plsc.load_gather(pm, [adr])
                plsc.store_scatter(pm, [adr], jnp.maximum(cm, val))
            return 0

        pl.loop(0, NPT, init_carry=0)(node)
        pltpu.sync_copy(ps, psum_hbm.at[pl.ds(w * PF, PF)])
        pltpu.sync_copy(pm, pmax_hbm.at[pl.ds(w * PF, PF)])

    return k(hflat, batch_pad)


def _node_ab(h, W1, b1, Din, PADN):
    BR = PADN // 8

    def body(h_ref, w_ref, b_ref, a_ref, bo_ref):
        wa = w_ref[0:Din, :]
        wb = w_ref[Din:2 * Din, :]
        hv = h_ref[...]
        a_ref[...] = jnp.dot(hv, wa - wb,
                             preferred_element_type=jnp.float32,
                             precision=lax.Precision.HIGHEST) + b_ref[...]
        hb = jnp.dot(hv, wb, preferred_element_type=jnp.float32,
                     precision=lax.Precision.HIGHEST)
        bo_ref[...] = jnp.concatenate(
            [hb, jnp.zeros((BR, F), jnp.float32)], axis=1)

    return pl.pallas_call(
        body,
        grid=(8,),
        in_specs=[
            pl.BlockSpec((BR, Din), lambda i: (i, 0)),
            pl.BlockSpec((2 * Din, F), lambda i: (0, 0)),
            pl.BlockSpec((1, F), lambda i: (0, 0)),
        ],
        out_specs=[
            pl.BlockSpec((BR, F), lambda i: (i, 0)),
            pl.BlockSpec((BR, 2 * F), lambda i: (i, 0)),
        ],
        out_shape=[
            jax.ShapeDtypeStruct((PADN, F), jnp.float32),
            jax.ShapeDtypeStruct((PADN, 2 * F), jnp.float32),
        ],
    )(h, W1, b1)


def _edge_mlp(pre, W2, b2, PADE):
    BR = 1280

    def body(p_ref, w_ref, b_ref, o_ref):
        mm = jnp.maximum(p_ref[...], 0.0)
        o_ref[...] = jnp.dot(mm, w_ref[...],
                             preferred_element_type=jnp.float32,
                             precision=lax.Precision.HIGHEST) + b_ref[...]

    return pl.pallas_call(
        body,
        grid=(PADE // BR,),
        in_specs=[
            pl.BlockSpec((BR, F), lambda i: (i, 0)),
            pl.BlockSpec((F, F), lambda i: (0, 0)),
            pl.BlockSpec((1, F), lambda i: (0, 0)),
        ],
        out_specs=pl.BlockSpec((BR, F), lambda i: (i, 0)),
        out_shape=jax.ShapeDtypeStruct((PADE, F), jnp.float32),
    )(pre, W2, b2)


def _classifier(psum, pmax, batch2d, cw1, cb1, cw2, cb2, cw3, cb3, G):
    NG = G + 1
    NBR, NBC = batch2d.shape

    def body(ps_ref, pm_ref, bt_ref, w1_ref, b1_ref, w2_ref, b2_ref, w3_ref,
             b3_ref, o_ref):
        s = jnp.sum(ps_ref[...], axis=0)[0:G, :]
        mx = jnp.max(pm_ref[...], axis=0)[0:G, :]
        mx = jnp.where(mx == NEG, 0.0, mx)
        counts = jnp.zeros((G, 1), jnp.float32)
        giota = lax.broadcasted_iota(jnp.int32, (G, NBC), 0)
        for i in range(NBR):
            bi = bt_ref[i:i + 1, :]
            oh = (jnp.broadcast_to(bi, (G, NBC)) == giota)
            counts = counts + jnp.sum(oh.astype(jnp.float32), axis=1,
                                      keepdims=True)
        x_mean = s / jnp.maximum(counts, 1.0)
        z = jnp.concatenate([x_mean, mx], axis=1)
        z = jnp.maximum(jnp.dot(z, w1_ref[...],
                                preferred_element_type=jnp.float32,
                                precision=lax.Precision.HIGHEST)
                        + b1_ref[...], 0.0)
        z = jnp.maximum(jnp.dot(z, w2_ref[...],
                                preferred_element_type=jnp.float32,
                                precision=lax.Precision.HIGHEST)
                        + b2_ref[...], 0.0)
        o_ref[...] = jnp.dot(z, w3_ref[...],
                             preferred_element_type=jnp.float32,
                             precision=lax.Precision.HIGHEST) + b3_ref[...]

    def full(shape):
        return pl.BlockSpec(shape, lambda: tuple(0 for _ in shape))

    return pl.pallas_call(
        body,
        in_specs=[
            full((TILES, NG, F)),
            full((TILES, NG, F)),
            full((NBR, NBC)),
            full((2 * F, F)),
            full((1, F)),
            full((F, F)),
            full((1, F)),
            full((F, 128)),
            full((1, 128)),
        ],
        out_specs=full((G, 128)),
        out_shape=jax.ShapeDtypeStruct((G, 128), jnp.float32),
    )(psum, pmax, batch2d, cw1, cb1, cw2, cb2, cw3, cb3)


def kernel(x, edge_index, batch, params):
    N, D = x.shape
    E = edge_index.shape[1]
    G = 64
    PADN = TILES * NPT
    PADE = E + 5120

    src = edge_index[0]
    dst = edge_index[1]
    xp = jnp.pad(x, ((0, PADN - N), (0, 0)))
    batch_pad = jnp.pad(batch, (0, PADN - N), constant_values=G)

    counts = _count(dst, E)
    dlocl, srcl = _compact(dst, src, counts, E, PADE)

    h = xp
    din = D
    for i in range(3):
        W1 = params[f"conv{i}_W1"]
        b1 = params[f"conv{i}_b1"].reshape(1, F)
        W2 = params[f"conv{i}_W2"]
        b2 = params[f"conv{i}_b2"].reshape(1, F)
        A, Bpad = _node_ab(h, W1, b1, din, PADN)
        pre = _edge_pre(A, Bpad, dlocl, srcl, counts, PADN, PADE)
        m = _edge_mlp(pre, W2, b2, PADE)
        hflat = _scatter_max(m, dlocl, counts, PADN, PADE)
        h = hflat.reshape(PADN, F)
        din = F

    psum, pmax = _pool_sc(hflat, batch_pad, G, PADN)
    NG = G + 1
    psum3 = psum.reshape(TILES, NG, F)
    pmax3 = pmax.reshape(TILES, NG, F)

    batch2d = batch_pad.reshape(16, PADN // 16)
    cw1 = params["cls_W1"]
    cb1 = params["cls_b1"].reshape(1, F)
    cw2 = params["cls_W2"]
    cb2 = params["cls_b2"].reshape(1, F)
    cw3 = jnp.pad(params["cls_W3"], ((0, 0), (0, 127)))
    cb3 = jnp.pad(params["cls_b3"].reshape(1, 1), ((0, 0), (0, 127)))
    out = _classifier(psum3, pmax3, batch2d, cw1, cb1, cw2, cb2, cw3, cb3, G)
    return out[:, 0]


# speedup vs baseline: 1.9096x; 1.2247x over previous
import functools

import jax
import jax.numpy as jnp
from jax import lax
from jax.experimental import pallas as pl
from jax.experimental.pallas import tpu as pltpu
from jax.experimental.pallas import tpu_sc as plsc

NC = 2
NS = 16
LN = 16
TILES = NC * NS

F = 64
NPT = 320
DUMROW = NPT
NEG = float("-inf")

_MESH = plsc.VectorSubcoreMesh(
    core_axis_name="c", subcore_axis_name="s", num_cores=NC, num_subcores=NS)
_CP = pltpu.CompilerParams(needs_layout_passes=False)

CH = 2560


def _wid():
    return lax.axis_index("s") * NC + lax.axis_index("c")


def _iota():
    return lax.iota(jnp.int32, LN)


def _kern(out_type, scratch):
    return functools.partial(pl.kernel, out_type=out_type, mesh=_MESH,
                             scratch_types=scratch, compiler_params=_CP)


def _my_start(counts_vm, w):
    c0 = plsc.load_gather(counts_vm, [_iota() * LN])
    c1 = plsc.load_gather(counts_vm, [(LN + _iota()) * LN])
    i = _iota()
    s = (jnp.sum(jnp.where(i < w, c0, 0))
         + jnp.sum(jnp.where(i + LN < w, c1, 0)))
    mynb = jnp.max(plsc.load_gather(counts_vm, [jnp.broadcast_to(w * LN, (LN,))]))
    return s, mynb


def _count(dst, E):
    @_kern(jax.ShapeDtypeStruct((TILES * LN,), jnp.int32),
           [pltpu.VMEM((CH,), jnp.int32), pltpu.VMEM((LN,), jnp.int32)])
    def k(dst_hbm, counts_hbm, dvm, cvb):
        w = _wid()
        lo = w * NPT
        hi = lo + NPT

        def chunk(ch, cv):
            pltpu.sync_copy(dst_hbm.at[pl.ds(ch * CH, CH)], dvm)
            for v in range(CH // LN):
                d = dvm[pl.ds(v * LN, LN)]
                cv = cv + ((d >= lo) & (d < hi)).astype(jnp.int32)
            return cv

        cv = pl.loop(0, E // CH, init_carry=jnp.zeros((LN,), jnp.int32))(chunk)
        nb = (jnp.sum(cv) + 127) // 128
        cvb[...] = jnp.broadcast_to(nb, (LN,))
        pltpu.sync_copy(cvb, counts_hbm.at[pl.ds(w * LN, LN)])

    return k(dst)


def _compact(dst, src, counts, E, PADE):
    STG = 4096

    @_kern([jax.ShapeDtypeStruct((PADE,), jnp.int32),
            jax.ShapeDtypeStruct((PADE,), jnp.int32)],
           [pltpu.VMEM((CH,), jnp.int32), pltpu.VMEM((CH,), jnp.int32),
            pltpu.VMEM((STG,), jnp.int32), pltpu.VMEM((STG,), jnp.int32),
            pltpu.VMEM((TILES * LN,), jnp.int32)])
    def k(dst_hbm, src_hbm, counts_hbm, dl_hbm, sl_hbm, dvm, svm, dstg, sstg,
          cvm):
        w = _wid()
        lo = w * NPT
        hi = lo + NPT
        iota = _iota()
        pltpu.sync_copy(counts_hbm, cvm)
        sb, _ = _my_start(cvm, w)
        base = pl.multiple_of(sb * 128, 128)

        def chunk(ch, carry):
            off, nfl = carry
            pltpu.sync_copy(dst_hbm.at[pl.ds(ch * CH, CH)], dvm)
            pltpu.sync_copy(src_hbm.at[pl.ds(ch * CH, CH)], svm)
            for v in range(CH // LN):
                d = dvm[pl.ds(v * LN, LN)]
                s = svm[pl.ds(v * LN, LN)]
                msk = (d >= lo) & (d < hi)
                cum = plsc.cumsum(msk.astype(jnp.int32))
                pos = (off + cum - 1) & (STG - 1)
                plsc.store_scatter(dstg, [pos], d - lo, mask=msk)
                plsc.store_scatter(sstg, [pos], s, mask=msk)
                off = off + jnp.sum(msk.astype(jnp.int32))

            def fl1024(kf, _):
                so = pl.multiple_of((kf * 1024) & (STG - 1), 1024)
                o = pl.multiple_of(base + kf * 1024, 128)
                pltpu.sync_copy(dstg.at[pl.ds(so, 1024)],
                                dl_hbm.at[pl.ds(o, 1024)])
                pltpu.sync_copy(sstg.at[pl.ds(so, 1024)],
                                sl_hbm.at[pl.ds(o, 1024)])
                return 0

            nfl_new = off // 1024
            pl.loop(nfl, nfl_new, init_carry=0)(fl1024)
            return off, nfl_new

        off, nfl = pl.loop(0, E // CH,
                           init_carry=(jnp.int32(0), jnp.int32(0)))(chunk)

        tgt = (off + 127) & (-128)
        dumd = jnp.full((LN,), DUMROW, jnp.int32)
        zero = jnp.zeros((LN,), jnp.int32)
        for kp in range(8):
            adr = off + kp * LN + iota
            m = adr < tgt
            pos = adr & (STG - 1)
            plsc.store_scatter(dstg, [pos], dumd, mask=m)
            plsc.store_scatter(sstg, [pos], zero, mask=m)

        def fl(q, _):
            eo = nfl * 1024 + q * 128
            so = pl.multiple_of(eo & (STG - 1), 128)
            o = pl.multiple_of(base + eo, 128)
            pltpu.sync_copy(dstg.at[pl.ds(so, 128)], dl_hbm.at[pl.ds(o, 128)])
            pltpu.sync_copy(sstg.at[pl.ds(so, 128)], sl_hbm.at[pl.ds(o, 128)])
            return 0

        pl.loop(0, (tgt - nfl * 1024) // 128, init_carry=0)(fl)

    return k(dst, src, counts)


def _edge_pre(A, Bpad, dlocl, srcl, counts, PADN, PADE):
    BT = 128

    @_kern(jax.ShapeDtypeStruct((PADE, F), jnp.float32),
           [pltpu.VMEM((NPT, F), jnp.float32),
            [pltpu.VMEM((BT,), jnp.int32)] * 2,
            [pltpu.VMEM((BT,), jnp.int32)] * 2,
            [pltpu.VMEM((BT, 2 * F), jnp.float32)] * 2,
            [pltpu.VMEM((BT, F), jnp.float32)] * 2,
            pltpu.VMEM((TILES * LN,), jnp.int32),
            [pltpu.SemaphoreType.DMA] * 2,
            [pltpu.SemaphoreType.DMA] * 2,
            [pltpu.SemaphoreType.DMA] * 2])
    def k(a_hbm, b_hbm, dl_hbm, sl_hbm, counts_hbm, pre_hbm, al, dlv, siv,
          brows, ost, cvm, sg, so, sd):
        w = _wid()
        iota = _iota()
        pltpu.sync_copy(counts_hbm, cvm)
        sb, mynb = _my_start(cvm, w)
        base = sb * 128
        pltpu.sync_copy(a_hbm.at[pl.ds(w * NPT, NPT), :], al)

        def stage(b, s):
            e0 = base + b * BT
            pltpu.sync_copy(dl_hbm.at[pl.ds(e0, BT)], dlv[s])
            pltpu.sync_copy(sl_hbm.at[pl.ds(e0, BT)], siv[s])
            pltpu.make_async_copy(b_hbm.at[siv[s]], brows[s], sg[s]).start()

        @pl.when(mynb > 0)
        def _():
            stage(jnp.int32(0), 0)

        def outer(t, _):
            for s in (0, 1):
                b = t * 2 + s

                @pl.when(b < mynb)
                def _():
                    nb_ = b + 1

                    @pl.when(nb_ < mynb)
                    def _():
                        stage(nb_, 1 - s)

                    pltpu.make_async_copy(b_hbm.at[siv[s]], brows[s],
                                          sg[s]).wait()

                    @pl.when(b >= 2)
                    def _():
                        pltpu.make_async_copy(
                            ost[s], pre_hbm.at[pl.ds(base, BT), :],
                            so[s]).wait()

                    def edge(j, _c):
                        for u in range(4):
                            jv = jnp.broadcast_to(j + u, (LN,))
                            dl = plsc.load_gather(dlv[s], [jv])
                            for q in range(F // LN):
                                ci = q * LN + iota
                                av = plsc.load_gather(al, [dl, ci])
                                bv = plsc.load_gather(brows[s], [jv, ci])
                                plsc.store_scatter(ost[s], [jv, ci], av + bv)
                        return 0

                    pl.loop(0, BT, init_carry=0, step=4)(edge)
                    e0 = base + b * BT
                    pltpu.make_async_copy(ost[s],
                                          pre_hbm.at[pl.ds(e0, BT), :],
                                          so[s]).start()
            return 0

        pl.loop(0, (mynb + 1) // 2, init_carry=0)(outer)
        for s in (0, 1):
            @pl.when(mynb > s)
            def _():
                pltpu.make_async_copy(ost[s], pre_hbm.at[pl.ds(base, BT), :],
                                      so[s]).wait()

    return k(A, Bpad, dlocl, srcl, counts)


def _scatter_max(m, dlocl, counts, PADN, PADE):
    BT = 128
    AF = (NPT + 1) * F

    @_kern(jax.ShapeDtypeStruct((PADN * F,), jnp.float32),
           [[pltpu.VMEM((BT,), jnp.int32)] * 2,
            [pltpu.VMEM((BT, F), jnp.float32)] * 2,
            pltpu.VMEM((AF,), jnp.float32),
            pltpu.VMEM((TILES * LN,), jnp.int32),
            [pltpu.SemaphoreType.DMA] * 2,
            [pltpu.SemaphoreType.DMA] * 2])
    def k(m_hbm, dl_hbm, counts_hbm, h_hbm, dlv, rows, acc, cvm, sr, sd):
        w = _wid()
        iota = _iota()
        pltpu.sync_copy(counts_hbm, cvm)
        sb, mynb = _my_start(cvm, w)
        base = sb * 128
        negv = jnp.full((LN,), NEG, jnp.float32)

        def stage(b, s):
            e0 = base + b * BT
            pltpu.make_async_copy(dl_hbm.at[pl.ds(e0, BT)], dlv[s],
                                  sd[s]).start()
            pltpu.make_async_copy(m_hbm.at[pl.ds(e0, BT), :], rows[s],
                                  sr[s]).start()

        @pl.when(mynb > 0)
        def _():
            stage(jnp.int32(0), 0)

        def init(i, _):
            plsc.store_scatter(acc, [i * LN + iota], negv)
            return 0

        pl.loop(0, AF // LN, init_carry=0)(init)

        def outer(t, _):
            for s in (0, 1):
                b = t * 2 + s

                @pl.when(b < mynb)
                def _():
                    nb_ = b + 1

                    @pl.when(nb_ < mynb)
                    def _():
                        stage(nb_, 1 - s)

                    pltpu.make_async_copy(dl_hbm.at[pl.ds(base, BT)], dlv[s],
                                          sd[s]).wait()
                    pltpu.make_async_copy(m_hbm.at[pl.ds(base, BT), :],
                                          rows[s], sr[s]).wait()

                    def edge(j, _c):
                        for u in range(4):
                            jv = jnp.broadcast_to(j + u, (LN,))
                            dl = plsc.load_gather(dlv[s], [jv])
                            for q in range(F // LN):
                                ci = q * LN + iota
                                rv = plsc.load_gather(rows[s], [jv, ci])
                                adr = dl * F + ci
                                av = plsc.load_gather(acc, [adr])
                                plsc.store_scatter(acc, [adr],
                                                   jnp.maximum(av, rv))
                        return 0

                    pl.loop(0, BT, init_carry=0, step=4)(edge)
            return 0

        pl.loop(0, (mynb + 1) // 2, init_carry=0)(outer)

        def fix(i, _):
            adr = i * LN + iota
            v = plsc.load_gather(acc, [adr])
            plsc.store_scatter(acc, [adr], jnp.where(v == NEG, 0.0, v))
            return 0

        pl.loop(0, NPT * F // LN, init_carry=0)(fix)
        pltpu.sync_copy(acc.at[pl.ds(0, NPT * F)],
                        h_hbm.at[pl.ds(w * NPT * F, NPT * F)])

    return k(m, dlocl, counts)


def _pool_sc(hflat, batch_pad, G, PADN):
    NG = G + 1
    PF = NG * F

    @_kern([jax.ShapeDtypeStruct((TILES * PF,), jnp.float32),
            jax.ShapeDtypeStruct((TILES * PF,), jnp.float32)],
           [pltpu.VMEM((NPT * F,), jnp.float32),
            pltpu.VMEM((NPT,), jnp.int32),
            pltpu.VMEM((PF,), jnp.float32),
            pltpu.VMEM((PF,), jnp.float32)])
    def k(h_hbm, b_hbm, psum_hbm, pmax_hbm, hv, bv, ps, pm):
        w = _wid()
        iota = _iota()
        pltpu.sync_copy(h_hbm.at[pl.ds(w * NPT * F, NPT * F)], hv)
        pltpu.sync_copy(b_hbm.at[pl.ds(w * NPT, NPT)], bv)
        negv = jnp.full((LN,), NEG, jnp.float32)
        zv = jnp.zeros((LN,), jnp.float32)

        def init(i, _):
            plsc.store_scatter(ps, [i * LN + iota], zv)
            plsc.store_scatter(pm, [i * LN + iota], negv)
            return 0

        pl.loop(0, PF // LN, init_carry=0)(init)

        def node(j, _c):
            jv = jnp.broadcast_to(j, (LN,))
            g = plsc.load_gather(bv, [jv])
            for q in range(F // LN):
                ci = q * LN + iota
                val = plsc.load_gather(hv, [jv * F + ci])
                adr = g * F + ci
                cs = plsc.load_gather(ps, [adr])
                plsc.store_scatter(ps, [adr], cs + val)
                cm = plsc.load_gather(pm, [adr])
                plsc.store_scatter(pm, [adr], jnp.maximum(cm, val))
            return 0

        pl.loop(0, NPT, init_carry=0)(node)
        pltpu.sync_copy(ps, psum_hbm.at[pl.ds(w * PF, PF)])
        pltpu.sync_copy(pm, pmax_hbm.at[pl.ds(w * PF, PF)])

    return k(hflat, batch_pad)


def _node_ab(h, W1, b1, Din, PADN):
    BR = PADN // 8

    def body(h_ref, w_ref, b_ref, a_ref, bo_ref):
        wa = w_ref[0:Din, :]
        wb = w_ref[Din:2 * Din, :]
        hv = h_ref[...]
        a_ref[...] = jnp.dot(hv, wa - wb,
                             preferred_element_type=jnp.float32,
                             precision=lax.Precision.HIGHEST) + b_ref[...]
        hb = jnp.dot(hv, wb, preferred_element_type=jnp.float32,
                     precision=lax.Precision.HIGHEST)
        bo_ref[...] = jnp.concatenate(
            [hb, jnp.zeros((BR, F), jnp.float32)], axis=1)

    return pl.pallas_call(
        body,
        grid=(8,),
        in_specs=[
            pl.BlockSpec((BR, Din), lambda i: (i, 0)),
            pl.BlockSpec((2 * Din, F), lambda i: (0, 0)),
            pl.BlockSpec((1, F), lambda i: (0, 0)),
        ],
        out_specs=[
            pl.BlockSpec((BR, F), lambda i: (i, 0)),
            pl.BlockSpec((BR, 2 * F), lambda i: (i, 0)),
        ],
        out_shape=[
            jax.ShapeDtypeStruct((PADN, F), jnp.float32),
            jax.ShapeDtypeStruct((PADN, 2 * F), jnp.float32),
        ],
    )(h, W1, b1)


def _edge_mlp(pre, W2, b2, PADE):
    BR = 1280

    def body(p_ref, w_ref, b_ref, o_ref):
        mm = jnp.maximum(p_ref[...], 0.0)
        o_ref[...] = jnp.dot(mm, w_ref[...],
                             preferred_element_type=jnp.float32,
                             precision=lax.Precision.HIGHEST) + b_ref[...]

    return pl.pallas_call(
        body,
        grid=(PADE // BR,),
        in_specs=[
            pl.BlockSpec((BR, F), lambda i: (i, 0)),
            pl.BlockSpec((F, F), lambda i: (0, 0)),
            pl.BlockSpec((1, F), lambda i: (0, 0)),
        ],
        out_specs=pl.BlockSpec((BR, F), lambda i: (i, 0)),
        out_shape=jax.ShapeDtypeStruct((PADE, F), jnp.float32),
    )(pre, W2, b2)


def _classifier(psum, pmax, batch2d, cw1, cb1, cw2, cb2, cw3, cb3, G):
    NG = G + 1
    NBR, NBC = batch2d.shape

    def body(ps_ref, pm_ref, bt_ref, w1_ref, b1_ref, w2_ref, b2_ref, w3_ref,
             b3_ref, o_ref):
        s = jnp.sum(ps_ref[...], axis=0)[0:G, :]
        mx = jnp.max(pm_ref[...], axis=0)[0:G, :]
        mx = jnp.where(mx == NEG, 0.0, mx)
        counts = jnp.zeros((G, 1), jnp.float32)
        giota = lax.broadcasted_iota(jnp.int32, (G, NBC), 0)
        for i in range(NBR):
            bi = bt_ref[i:i + 1, :]
            oh = (jnp.broadcast_to(bi, (G, NBC)) == giota)
            counts = counts + jnp.sum(oh.astype(jnp.float32), axis=1,
                                      keepdims=True)
        x_mean = s / jnp.maximum(counts, 1.0)
        z = jnp.concatenate([x_mean, mx], axis=1)
        z = jnp.maximum(jnp.dot(z, w1_ref[...],
                                preferred_element_type=jnp.float32,
                                precision=lax.Precision.HIGHEST)
                        + b1_ref[...], 0.0)
        z = jnp.maximum(jnp.dot(z, w2_ref[...],
                                preferred_element_type=jnp.float32,
                                precision=lax.Precision.HIGHEST)
                        + b2_ref[...], 0.0)
        o_ref[...] = jnp.dot(z, w3_ref[...],
                             preferred_element_type=jnp.float32,
                             precision=lax.Precision.HIGHEST) + b3_ref[...]

    def full(shape):
        return pl.BlockSpec(shape, lambda: tuple(0 for _ in shape))

    return pl.pallas_call(
        body,
        in_specs=[
            full((TILES, NG, F)),
            full((TILES, NG, F)),
            full((NBR, NBC)),
            full((2 * F, F)),
            full((1, F)),
            full((F, F)),
            full((1, F)),
            full((F, 128)),
            full((1, 128)),
        ],
        out_specs=full((G, 128)),
        out_shape=jax.ShapeDtypeStruct((G, 128), jnp.float32),
    )(psum, pmax, batch2d, cw1, cb1, cw2, cb2, cw3, cb3)


def kernel(x, edge_index, batch, params):
    N, D = x.shape
    E = edge_index.shape[1]
    G = 64
    PADN = TILES * NPT
    PADE = E + 5120

    src = edge_index[0]
    dst = edge_index[1]
    xp = jnp.pad(x, ((0, PADN - N), (0, 0)))
    batch_pad = jnp.pad(batch, (0, PADN - N), constant_values=G)

    counts = _count(dst, E)
    dlocl, srcl = _compact(dst, src, counts, E, PADE)

    h = xp
    din = D
    for i in range(3):
        W1 = params[f"conv{i}_W1"]
        b1 = params[f"conv{i}_b1"].reshape(1, F)
        W2 = params[f"conv{i}_W2"]
        b2 = params[f"conv{i}_b2"].reshape(1, F)
        A, Bpad = _node_ab(h, W1, b1, din, PADN)
        pre = _edge_pre(A, Bpad, dlocl, srcl, counts, PADN, PADE)
        m = _edge_mlp(pre, W2, b2, PADE)
        hflat = _scatter_max(m, dlocl, counts, PADN, PADE)
        h = hflat.reshape(PADN, F)
        din = F

    psum, pmax = _pool_sc(hflat, batch_pad, G, PADN)
    NG = G + 1
    psum3 = psum.reshape(TILES, NG, F)
    pmax3 = pmax.reshape(TILES, NG, F)

    batch2d = batch_pad.reshape(16, PADN // 16)
    cw1 = params["cls_W1"]
    cb1 = params["cls_b1"].reshape(1, F)
    cw2 = params["cls_W2"]
    cb2 = params["cls_b2"].reshape(1, F)
    cw3 = jnp.pad(params["cls_W3"], ((0, 0), (0, 127)))
    cb3 = jnp.pad(params["cls_b3"].reshape(1, 1), ((0, 0), (0, 127)))
    out = _classifier(psum3, pmax3, batch2d, cw1, cb1, cw2, cb2, cw3, cb3, G)
    return out[:, 0]
